# Initial kernel scaffold; baseline (speedup 1.0000x reference)
import functools

import jax
import jax.numpy as jnp
from jax import lax
from jax.experimental import pallas as pl
from jax.experimental.pallas import tpu as pltpu
from jax.experimental.pallas import tpu_sc as plsc

N = 10000
NP = 10240
E = 320000
IN_DIM = 128
HID = 64

NSC = 2
NSUB = 16
NW = NSC * NSUB
CH = 128
CPW = -(-E // (NW * CH))
EP = NW * CH * CPW
RPS = NP // NSUB

_DUMMY_DST = N


@functools.lru_cache(maxsize=None)
def _make_propagate(D):
  mesh = plsc.VectorSubcoreMesh(core_axis_name="c", subcore_axis_name="s")

  @functools.partial(
      pl.kernel,
      out_type=jax.ShapeDtypeStruct((NSC, NP, D), jnp.float32),
      mesh=mesh,
      scratch_types=[
          pltpu.VMEM((CPW, CH), jnp.int32),
          pltpu.VMEM((CPW, CH), jnp.int32),
          pltpu.VMEM((CH, D), jnp.float32),
          pltpu.VMEM_SHARED((NP, D), jnp.float32),
          pltpu.SemaphoreType.DMA,
      ],
  )
  def prop(src_hbm, dst_hbm, g_hbm, zeros_hbm, out_hbm,
           src_v, dst_v, rows_v, acc, isem):
    c = lax.axis_index("c")
    s = lax.axis_index("s")
    w = s * NSC + c

    pltpu.sync_copy(zeros_hbm, acc.at[pl.ds(s * RPS, RPS)])
    pltpu.async_copy(src_hbm.at[pl.ds(w * CPW, CPW)], src_v, isem).wait()
    pltpu.async_copy(dst_hbm.at[pl.ds(w * CPW, CPW)], dst_v, isem).wait()
    plsc.subcore_barrier()

    @pl.loop(0, CPW)
    def _(j):
      pltpu.sync_copy(g_hbm.at[src_v.at[j]], rows_v)
      pltpu.sync_copy(rows_v, acc.at[dst_v.at[j]], add=True)

    plsc.subcore_barrier()
    pltpu.sync_copy(acc.at[pl.ds(s * RPS, RPS)],
                    out_hbm.at[c, pl.ds(s * RPS, RPS)])

  return prop


B = 512
GRID = (NP // B,)


def _row_spec(d):
  return pl.BlockSpec((B, d), lambda i: (i, 0))


def _part_spec(d):
  return pl.BlockSpec((NSC, B, d), lambda i: (0, i, 0))


def _full_spec(a, b):
  return pl.BlockSpec((a, b), lambda i: (0, 0))


def _stage1(x, w1, degp):
  def body(x_ref, w_ref, dp_ref, g1_ref, dv_ref):
    dp = dp_ref[0] + dp_ref[1]
    dinv = lax.rsqrt(1.0 + dp[:, 0:1])
    xw = jnp.dot(x_ref[...], w_ref[...], preferred_element_type=jnp.float32)
    g1_ref[...] = dinv * xw
    dv_ref[...] = jnp.broadcast_to(dinv, (B, 8))

  return pl.pallas_call(
      body,
      grid=GRID,
      in_specs=[_row_spec(IN_DIM), _full_spec(IN_DIM, HID), _part_spec(16)],
      out_specs=[_row_spec(HID), _row_spec(8)],
      out_shape=[jax.ShapeDtypeStruct((NP, HID), jnp.float32),
                 jax.ShapeDtypeStruct((NP, 8), jnp.float32)],
  )(x, w1, degp)


def _stage2(p1, g1, dv, b1, w2):
  def body(p_ref, g_ref, dv_ref, b_ref, w_ref, g2_ref):
    dinv = dv_ref[:, 0:1]
    h1 = jnp.maximum(dinv * (p_ref[0] + p_ref[1] + g_ref[...]) + b_ref[...],
                     0.0)
    g2_ref[...] = dinv * jnp.dot(h1, w_ref[...],
                                 preferred_element_type=jnp.float32)

  return pl.pallas_call(
      body,
      grid=GRID,
      in_specs=[_part_spec(HID), _row_spec(HID), _row_spec(8),
                _full_spec(1, HID), _full_spec(HID, HID)],
      out_specs=_row_spec(HID),
      out_shape=jax.ShapeDtypeStruct((NP, HID), jnp.float32),
  )(p1, g1, dv, b1, w2)


def _stage3(p2, g2, dv, b2, aw1):
  def body(p_ref, g_ref, dv_ref, b_ref, w_ref, g3_ref):
    dinv = dv_ref[:, 0:1]
    h = dinv * (p_ref[0] + p_ref[1] + g_ref[...]) + b_ref[...]
    hw = jnp.dot(h, w_ref[...], preferred_element_type=jnp.float32)
    g3_ref[...] = jnp.concatenate([dinv * hw, dinv * h], axis=1)

  return pl.pallas_call(
      body,
      grid=GRID,
      in_specs=[_part_spec(HID), _row_spec(HID), _row_spec(8),
                _full_spec(1, HID), _full_spec(HID, HID)],
      out_specs=_row_spec(2 * HID),
      out_shape=jax.ShapeDtypeStruct((NP, 2 * HID), jnp.float32),
  )(p2, g2, dv, b2, aw1)


def _stage4(p3, g3, dv, ab1, sw1, sb1):
  def body(p_ref, g_ref, dv_ref, ab_ref, sw_ref, sb_ref, g4_ref, h_ref):
    dinv = dv_ref[:, 0:1]
    pa = p_ref[0, :, :HID] + p_ref[1, :, :HID]
    x1 = jnp.maximum(dinv * (pa + g_ref[:, :HID]) + ab_ref[...], 0.0)
    g4_ref[...] = dinv * x1
    pb = p_ref[0, :, HID:] + p_ref[1, :, HID:]
    hpre = dinv * (pb + g_ref[:, HID:])
    h_ref[...] = jnp.dot(hpre, sw_ref[...],
                         preferred_element_type=jnp.float32) + sb_ref[...]

  return pl.pallas_call(
      body,
      grid=GRID,
      in_specs=[_part_spec(2 * HID), _row_spec(2 * HID), _row_spec(8),
                _full_spec(1, HID), _full_spec(HID, IN_DIM),
                _full_spec(1, IN_DIM)],
      out_specs=[_row_spec(HID), _row_spec(IN_DIM)],
      out_shape=[jax.ShapeDtypeStruct((NP, HID), jnp.float32),
                 jax.ShapeDtypeStruct((NP, IN_DIM), jnp.float32)],
  )(p3, g3, dv, ab1, sw1, sb1)


def _stage5(p4, g4, dv, aw2, ab2):
  def body(p_ref, g_ref, dv_ref, w_ref, b_ref, x_ref):
    dinv = dv_ref[:, 0:1]
    xpre = dinv * (p_ref[0] + p_ref[1] + g_ref[...])
    x_ref[...] = jnp.dot(xpre, w_ref[...],
                         preferred_element_type=jnp.float32) + b_ref[...]

  return pl.pallas_call(
      body,
      grid=GRID,
      in_specs=[_part_spec(HID), _row_spec(HID), _row_spec(8),
                _full_spec(HID, IN_DIM), _full_spec(1, IN_DIM)],
      out_specs=_row_spec(IN_DIM),
      out_shape=jax.ShapeDtypeStruct((NP, IN_DIM), jnp.float32),
  )(p4, g4, dv, aw2, ab2)


def _adj(h):
  BM, BN = 512, 1024

  def body(a_ref, b_ref, o_ref):
    o_ref[...] = lax.dot_general(
        a_ref[...], b_ref[...], (((1,), (1,)), ((), ())),
        preferred_element_type=jnp.float32)

  return pl.pallas_call(
      body,
      grid=(-(-N // BM), -(-N // BN)),
      in_specs=[pl.BlockSpec((BM, IN_DIM), lambda i, j: (i, 0)),
                pl.BlockSpec((BN, IN_DIM), lambda i, j: (j, 0))],
      out_specs=pl.BlockSpec((BM, BN), lambda i, j: (i, j)),
      out_shape=jax.ShapeDtypeStruct((N, N), jnp.float32),
  )(h, h)


def kernel(x, edge_index, enc_W1, enc_b1, enc_W2, enc_b2,
           attr_W1, attr_b1, attr_W2, attr_b2, struct_W1, struct_b1):
  src = edge_index[0]
  dst = edge_index[1]
  pad = EP - E
  src_p = jnp.concatenate([src, jnp.zeros((pad,), jnp.int32)])
  src_p = src_p.reshape(NW * CPW, CH)
  dst_p = jnp.concatenate([dst, jnp.full((pad,), _DUMMY_DST, jnp.int32)])
  dst_p = dst_p.reshape(NW * CPW, CH)

  x_p = jnp.pad(x, ((0, NP - N), (0, 0)))
  ones16 = jnp.ones((NP, 16), jnp.float32)
  z16 = jnp.zeros((RPS, 16), jnp.float32)
  z64 = jnp.zeros((RPS, HID), jnp.float32)
  z128 = jnp.zeros((RPS, 2 * HID), jnp.float32)
  b1 = enc_b1.reshape(1, HID)
  b2 = enc_b2.reshape(1, HID)
  ab1 = attr_b1.reshape(1, HID)
  ab2 = attr_b2.reshape(1, IN_DIM)
  sb1 = struct_b1.reshape(1, IN_DIM)

  degp = _make_propagate(16)(src_p, dst_p, ones16, z16)

  g1, dv = _stage1(x_p, enc_W1, degp)
  p1 = _make_propagate(HID)(src_p, dst_p, g1, z64)
  g2 = _stage2(p1, g1, dv, b1, enc_W2)
  p2 = _make_propagate(HID)(src_p, dst_p, g2, z64)
  g3 = _stage3(p2, g2, dv, b2, attr_W1)
  p3 = _make_propagate(2 * HID)(src_p, dst_p, g3, z128)
  g4, h_ = _stage4(p3, g3, dv, attr_b1, struct_W1, struct_b1)
  p4 = _make_propagate(HID)(src_p, dst_p, g4, z64)
  x_ = _stage5(p4, g4, dv, attr_W2, attr_b2)

  adj = _adj(h_)
  return (x_[:N], adj)


# speedup vs baseline: 8.9110x; 8.9110x over previous
import functools

import jax
import jax.numpy as jnp
from jax import lax
from jax.experimental import pallas as pl
from jax.experimental.pallas import tpu as pltpu
from jax.experimental.pallas import tpu_sc as plsc

N = 10000
NP = 10240
E = 320000
IN_DIM = 128
HID = 64

NSC = 2
NSUB = 16
NW = NSC * NSUB
CH = 128
CPW = 8 * -(-E // (NW * CH * 8))
EP = NW * CH * CPW
RPS = NP // NSUB

_DUMMY_DST = N


@functools.lru_cache(maxsize=None)
def _make_propagate(D):
  mesh = plsc.VectorSubcoreMesh(core_axis_name="c", subcore_axis_name="s")

  @functools.partial(
      pl.kernel,
      out_type=jax.ShapeDtypeStruct((NSC, NP, D), jnp.float32),
      mesh=mesh,
      compiler_params=pltpu.CompilerParams(use_tc_tiling_on_sc=False),
      scratch_types=[
          pltpu.VMEM((CPW, CH), jnp.int32),
          pltpu.VMEM((CPW, CH), jnp.int32),
          pltpu.VMEM((CH, D), jnp.float32),
          pltpu.VMEM_SHARED((NP, D), jnp.float32),
          pltpu.SemaphoreType.DMA,
      ],
  )
  def prop(src_hbm, dst_hbm, g_hbm, zeros_hbm, out_hbm,
           src_v, dst_v, rows_v, acc, isem):
    c = lax.axis_index("c")
    s = lax.axis_index("s")
    w = s * NSC + c

    pltpu.sync_copy(zeros_hbm, acc.at[pl.ds(s * RPS, RPS)])
    pltpu.async_copy(src_hbm.at[pl.ds(w * CPW, CPW)], src_v, isem).wait()
    pltpu.async_copy(dst_hbm.at[pl.ds(w * CPW, CPW)], dst_v, isem).wait()
    plsc.subcore_barrier()

    @pl.loop(0, CPW)
    def _(j):
      pltpu.sync_copy(g_hbm.at[src_v.at[j]], rows_v)
      pltpu.sync_copy(rows_v, acc.at[dst_v.at[j]], add=True)

    plsc.subcore_barrier()
    pltpu.sync_copy(acc.at[pl.ds(s * RPS, RPS)],
                    out_hbm.at[c, pl.ds(s * RPS, RPS)])

  return prop


B = 512
GRID = (NP // B,)


def _row_spec(d):
  return pl.BlockSpec((B, d), lambda i: (i, 0))


def _part_spec(d):
  return pl.BlockSpec((NSC, B, d), lambda i: (0, i, 0))


def _full_spec(a, b):
  return pl.BlockSpec((a, b), lambda i: (0, 0))


def _stage1(x, w1, degp):
  def body(x_ref, w_ref, dp_ref, g1_ref, dv_ref):
    dp = dp_ref[0] + dp_ref[1]
    dinv = lax.rsqrt(1.0 + dp[:, 0:1])
    xw = jnp.dot(x_ref[...], w_ref[...], preferred_element_type=jnp.float32)
    g1_ref[...] = dinv * xw
    dv_ref[...] = jnp.broadcast_to(dinv, (B, 8))

  return pl.pallas_call(
      body,
      grid=GRID,
      in_specs=[_row_spec(IN_DIM), _full_spec(IN_DIM, HID), _part_spec(16)],
      out_specs=[_row_spec(HID), _row_spec(8)],
      out_shape=[jax.ShapeDtypeStruct((NP, HID), jnp.float32),
                 jax.ShapeDtypeStruct((NP, 8), jnp.float32)],
  )(x, w1, degp)


def _stage2(p1, g1, dv, b1, w2):
  def body(p_ref, g_ref, dv_ref, b_ref, w_ref, g2_ref):
    dinv = dv_ref[:, 0:1]
    h1 = jnp.maximum(dinv * (p_ref[0] + p_ref[1] + g_ref[...]) + b_ref[...],
                     0.0)
    g2_ref[...] = dinv * jnp.dot(h1, w_ref[...],
                                 preferred_element_type=jnp.float32)

  return pl.pallas_call(
      body,
      grid=GRID,
      in_specs=[_part_spec(HID), _row_spec(HID), _row_spec(8),
                _full_spec(1, HID), _full_spec(HID, HID)],
      out_specs=_row_spec(HID),
      out_shape=jax.ShapeDtypeStruct((NP, HID), jnp.float32),
  )(p1, g1, dv, b1, w2)


def _stage3(p2, g2, dv, b2, aw1):
  def body(p_ref, g_ref, dv_ref, b_ref, w_ref, g3_ref):
    dinv = dv_ref[:, 0:1]
    h = dinv * (p_ref[0] + p_ref[1] + g_ref[...]) + b_ref[...]
    hw = jnp.dot(h, w_ref[...], preferred_element_type=jnp.float32)
    g3_ref[...] = jnp.concatenate([dinv * hw, dinv * h], axis=1)

  return pl.pallas_call(
      body,
      grid=GRID,
      in_specs=[_part_spec(HID), _row_spec(HID), _row_spec(8),
                _full_spec(1, HID), _full_spec(HID, HID)],
      out_specs=_row_spec(2 * HID),
      out_shape=jax.ShapeDtypeStruct((NP, 2 * HID), jnp.float32),
  )(p2, g2, dv, b2, aw1)


def _stage4(p3, g3, dv, ab1, sw1, sb1):
  def body(p_ref, g_ref, dv_ref, ab_ref, sw_ref, sb_ref, g4_ref, h_ref):
    dinv = dv_ref[:, 0:1]
    pa = p_ref[0, :, :HID] + p_ref[1, :, :HID]
    x1 = jnp.maximum(dinv * (pa + g_ref[:, :HID]) + ab_ref[...], 0.0)
    g4_ref[...] = dinv * x1
    pb = p_ref[0, :, HID:] + p_ref[1, :, HID:]
    hpre = dinv * (pb + g_ref[:, HID:])
    h_ref[...] = jnp.dot(hpre, sw_ref[...],
                         preferred_element_type=jnp.float32) + sb_ref[...]

  return pl.pallas_call(
      body,
      grid=GRID,
      in_specs=[_part_spec(2 * HID), _row_spec(2 * HID), _row_spec(8),
                _full_spec(1, HID), _full_spec(HID, IN_DIM),
                _full_spec(1, IN_DIM)],
      out_specs=[_row_spec(HID), _row_spec(IN_DIM)],
      out_shape=[jax.ShapeDtypeStruct((NP, HID), jnp.float32),
                 jax.ShapeDtypeStruct((NP, IN_DIM), jnp.float32)],
  )(p3, g3, dv, ab1, sw1, sb1)


def _stage5(p4, g4, dv, aw2, ab2):
  def body(p_ref, g_ref, dv_ref, w_ref, b_ref, x_ref):
    dinv = dv_ref[:, 0:1]
    xpre = dinv * (p_ref[0] + p_ref[1] + g_ref[...])
    x_ref[...] = jnp.dot(xpre, w_ref[...],
                         preferred_element_type=jnp.float32) + b_ref[...]

  return pl.pallas_call(
      body,
      grid=GRID,
      in_specs=[_part_spec(HID), _row_spec(HID), _row_spec(8),
                _full_spec(HID, IN_DIM), _full_spec(1, IN_DIM)],
      out_specs=_row_spec(IN_DIM),
      out_shape=jax.ShapeDtypeStruct((NP, IN_DIM), jnp.float32),
  )(p4, g4, dv, aw2, ab2)


def _adj(h):
  BM, BN = 512, 1024

  def body(a_ref, b_ref, o_ref):
    o_ref[...] = lax.dot_general(
        a_ref[...], b_ref[...], (((1,), (1,)), ((), ())),
        preferred_element_type=jnp.float32)

  return pl.pallas_call(
      body,
      grid=(-(-N // BM), -(-N // BN)),
      in_specs=[pl.BlockSpec((BM, IN_DIM), lambda i, j: (i, 0)),
                pl.BlockSpec((BN, IN_DIM), lambda i, j: (j, 0))],
      out_specs=pl.BlockSpec((BM, BN), lambda i, j: (i, j)),
      out_shape=jax.ShapeDtypeStruct((N, N), jnp.float32),
  )(h, h)


def kernel(x, edge_index, enc_W1, enc_b1, enc_W2, enc_b2,
           attr_W1, attr_b1, attr_W2, attr_b2, struct_W1, struct_b1):
  src = edge_index[0]
  dst = edge_index[1]
  pad = EP - E
  src_p = jnp.concatenate([src, jnp.zeros((pad,), jnp.int32)])
  src_p = src_p.reshape(NW * CPW, CH)
  dst_p = jnp.concatenate([dst, jnp.full((pad,), _DUMMY_DST, jnp.int32)])
  dst_p = dst_p.reshape(NW * CPW, CH)

  x_p = jnp.pad(x, ((0, NP - N), (0, 0)))
  ones16 = jnp.ones((NP, 16), jnp.float32)
  z16 = jnp.zeros((RPS, 16), jnp.float32)
  z64 = jnp.zeros((RPS, HID), jnp.float32)
  z128 = jnp.zeros((RPS, 2 * HID), jnp.float32)
  b1 = enc_b1.reshape(1, HID)
  b2 = enc_b2.reshape(1, HID)
  ab1 = attr_b1.reshape(1, HID)
  ab2 = attr_b2.reshape(1, IN_DIM)
  sb1 = struct_b1.reshape(1, IN_DIM)

  degp = _make_propagate(16)(src_p, dst_p, ones16, z16)

  g1, dv = _stage1(x_p, enc_W1, degp)
  p1 = _make_propagate(HID)(src_p, dst_p, g1, z64)
  g2 = _stage2(p1, g1, dv, b1, enc_W2)
  p2 = _make_propagate(HID)(src_p, dst_p, g2, z64)
  g3 = _stage3(p2, g2, dv, b2, attr_W1)
  p3 = _make_propagate(2 * HID)(src_p, dst_p, g3, z128)
  g4, h_ = _stage4(p3, g3, dv, ab1, struct_W1, sb1)
  p4 = _make_propagate(HID)(src_p, dst_p, g4, z64)
  x_ = _stage5(p4, g4, dv, attr_W2, ab2)

  adj = _adj(h_)
  return (x_[:N], adj)


# speedup vs baseline: 19.9620x; 2.2402x over previous
import functools

import jax
import jax.numpy as jnp
from jax import lax
from jax.experimental import pallas as pl
from jax.experimental.pallas import tpu as pltpu
from jax.experimental.pallas import tpu_sc as plsc

N = 10000
NP = 10240
E = 320000
IN_DIM = 128
HID = 64

NSC = 2
NSUB = 16
NW = NSC * NSUB
CH = 128
CPW = 8 * -(-E // (NW * CH * 8))
EP = NW * CH * CPW
RPS = NP // NSUB

_DUMMY_DST = N


@functools.lru_cache(maxsize=None)
def _make_propagate(D):
  mesh = plsc.VectorSubcoreMesh(core_axis_name="c", subcore_axis_name="s")
  nbuf = 8

  @functools.partial(
      pl.kernel,
      out_type=jax.ShapeDtypeStruct((NSC, NP, D), jnp.float32),
      mesh=mesh,
      compiler_params=pltpu.CompilerParams(use_tc_tiling_on_sc=False),
      scratch_types=[
          pltpu.VMEM((CPW, CH), jnp.int32),
          pltpu.VMEM((CPW, CH), jnp.int32),
          pltpu.VMEM((nbuf, CH, D), jnp.float32),
          pltpu.VMEM_SHARED((NP, D), jnp.float32),
          pltpu.SemaphoreType.DMA,
          pltpu.SemaphoreType.DMA((nbuf,)),
          pltpu.SemaphoreType.DMA,
      ],
  )
  def prop(src_hbm, dst_hbm, g_hbm, zeros_hbm, out_hbm,
           src_v, dst_v, rows_v, acc, isem, gsems, ssem):
    c = lax.axis_index("c")
    s = lax.axis_index("s")
    w = s * NSC + c

    pltpu.sync_copy(zeros_hbm, acc.at[pl.ds(s * RPS, RPS)])
    pltpu.async_copy(src_hbm.at[pl.ds(w * CPW, CPW)], src_v, isem).wait()
    pltpu.async_copy(dst_hbm.at[pl.ds(w * CPW, CPW)], dst_v, isem).wait()
    plsc.subcore_barrier()

    @pl.loop(0, CPW, step=nbuf)
    def _(j):
      for b in range(nbuf):
        pltpu.async_copy(g_hbm.at[src_v.at[j + b]], rows_v.at[b],
                         gsems.at[b])
      for b in range(nbuf):
        pltpu.make_async_copy(g_hbm.at[src_v.at[j + b]], rows_v.at[b],
                              gsems.at[b]).wait()
        pltpu.async_copy(rows_v.at[b], acc.at[dst_v.at[j + b]], ssem,
                         add=True)
      for b in range(nbuf):
        pltpu.make_async_copy(rows_v.at[b], acc.at[dst_v.at[j + b]],
                              ssem).wait()

    plsc.subcore_barrier()
    pltpu.sync_copy(acc.at[pl.ds(s * RPS, RPS)],
                    out_hbm.at[c, pl.ds(s * RPS, RPS)])

  return prop


B = 512
GRID = (NP // B,)


def _row_spec(d):
  return pl.BlockSpec((B, d), lambda i: (i, 0))


def _part_spec(d):
  return pl.BlockSpec((NSC, B, d), lambda i: (0, i, 0))


def _full_spec(a, b):
  return pl.BlockSpec((a, b), lambda i: (0, 0))


def _stage1(x, w1, degp):
  def body(x_ref, w_ref, dp_ref, g1_ref, dv_ref):
    dp = dp_ref[0] + dp_ref[1]
    dinv = lax.rsqrt(1.0 + dp[:, 0:1])
    xw = jnp.dot(x_ref[...], w_ref[...], preferred_element_type=jnp.float32)
    g1_ref[...] = dinv * xw
    dv_ref[...] = jnp.broadcast_to(dinv, (B, 8))

  return pl.pallas_call(
      body,
      grid=GRID,
      in_specs=[_row_spec(IN_DIM), _full_spec(IN_DIM, HID), _part_spec(16)],
      out_specs=[_row_spec(HID), _row_spec(8)],
      out_shape=[jax.ShapeDtypeStruct((NP, HID), jnp.float32),
                 jax.ShapeDtypeStruct((NP, 8), jnp.float32)],
  )(x, w1, degp)


def _stage2(p1, g1, dv, b1, w2):
  def body(p_ref, g_ref, dv_ref, b_ref, w_ref, g2_ref):
    dinv = dv_ref[:, 0:1]
    h1 = jnp.maximum(dinv * (p_ref[0] + p_ref[1] + g_ref[...]) + b_ref[...],
                     0.0)
    g2_ref[...] = dinv * jnp.dot(h1, w_ref[...],
                                 preferred_element_type=jnp.float32)

  return pl.pallas_call(
      body,
      grid=GRID,
      in_specs=[_part_spec(HID), _row_spec(HID), _row_spec(8),
                _full_spec(1, HID), _full_spec(HID, HID)],
      out_specs=_row_spec(HID),
      out_shape=jax.ShapeDtypeStruct((NP, HID), jnp.float32),
  )(p1, g1, dv, b1, w2)


def _stage3(p2, g2, dv, b2, aw1):
  def body(p_ref, g_ref, dv_ref, b_ref, w_ref, g3a_ref, g3b_ref):
    dinv = dv_ref[:, 0:1]
    h = dinv * (p_ref[0] + p_ref[1] + g_ref[...]) + b_ref[...]
    hw = jnp.dot(h, w_ref[...], preferred_element_type=jnp.float32)
    g3a_ref[...] = dinv * hw
    g3b_ref[...] = dinv * h

  return pl.pallas_call(
      body,
      grid=GRID,
      in_specs=[_part_spec(HID), _row_spec(HID), _row_spec(8),
                _full_spec(1, HID), _full_spec(HID, HID)],
      out_specs=[_row_spec(HID), _row_spec(HID)],
      out_shape=[jax.ShapeDtypeStruct((NP, HID), jnp.float32),
                 jax.ShapeDtypeStruct((NP, HID), jnp.float32)],
  )(p2, g2, dv, b2, aw1)


def _stage4(p3a, p3b, g3a, g3b, dv, ab1, sw1, sb1):
  def body(pa_ref, pb_ref, ga_ref, gb_ref, dv_ref, ab_ref, sw_ref, sb_ref,
           g4_ref, h_ref):
    dinv = dv_ref[:, 0:1]
    pa = pa_ref[0] + pa_ref[1]
    x1 = jnp.maximum(dinv * (pa + ga_ref[...]) + ab_ref[...], 0.0)
    g4_ref[...] = dinv * x1
    pb = pb_ref[0] + pb_ref[1]
    hpre = dinv * (pb + gb_ref[...])
    h_ref[...] = jnp.dot(hpre, sw_ref[...],
                         preferred_element_type=jnp.float32) + sb_ref[...]

  return pl.pallas_call(
      body,
      grid=GRID,
      in_specs=[_part_spec(HID), _part_spec(HID), _row_spec(HID),
                _row_spec(HID), _row_spec(8),
                _full_spec(1, HID), _full_spec(HID, IN_DIM),
                _full_spec(1, IN_DIM)],
      out_specs=[_row_spec(HID), _row_spec(IN_DIM)],
      out_shape=[jax.ShapeDtypeStruct((NP, HID), jnp.float32),
                 jax.ShapeDtypeStruct((NP, IN_DIM), jnp.float32)],
  )(p3a, p3b, g3a, g3b, dv, ab1, sw1, sb1)


def _stage5(p4, g4, dv, aw2, ab2):
  def body(p_ref, g_ref, dv_ref, w_ref, b_ref, x_ref):
    dinv = dv_ref[:, 0:1]
    xpre = dinv * (p_ref[0] + p_ref[1] + g_ref[...])
    x_ref[...] = jnp.dot(xpre, w_ref[...],
                         preferred_element_type=jnp.float32) + b_ref[...]

  return pl.pallas_call(
      body,
      grid=GRID,
      in_specs=[_part_spec(HID), _row_spec(HID), _row_spec(8),
                _full_spec(HID, IN_DIM), _full_spec(1, IN_DIM)],
      out_specs=_row_spec(IN_DIM),
      out_shape=jax.ShapeDtypeStruct((NP, IN_DIM), jnp.float32),
  )(p4, g4, dv, aw2, ab2)


def _adj(h):
  BM, BN = 512, 1024

  def body(a_ref, b_ref, o_ref):
    o_ref[...] = lax.dot_general(
        a_ref[...], b_ref[...], (((1,), (1,)), ((), ())),
        preferred_element_type=jnp.float32)

  return pl.pallas_call(
      body,
      grid=(-(-N // BM), -(-N // BN)),
      in_specs=[pl.BlockSpec((BM, IN_DIM), lambda i, j: (i, 0)),
                pl.BlockSpec((BN, IN_DIM), lambda i, j: (j, 0))],
      out_specs=pl.BlockSpec((BM, BN), lambda i, j: (i, j)),
      out_shape=jax.ShapeDtypeStruct((N, N), jnp.float32),
  )(h, h)


def kernel(x, edge_index, enc_W1, enc_b1, enc_W2, enc_b2,
           attr_W1, attr_b1, attr_W2, attr_b2, struct_W1, struct_b1):
  src = edge_index[0]
  dst = edge_index[1]
  pad = EP - E
  spread = jnp.arange(pad, dtype=jnp.int32)
  src_p = jnp.concatenate([src, spread % N])
  src_p = src_p.reshape(NW * CPW, CH)
  dst_p = jnp.concatenate([dst, _DUMMY_DST + spread % (NP - N)])
  dst_p = dst_p.reshape(NW * CPW, CH)

  x_p = jnp.pad(x, ((0, NP - N), (0, 0)))
  ones16 = jnp.ones((NP, 16), jnp.float32)
  z16 = jnp.zeros((RPS, 16), jnp.float32)
  z64 = jnp.zeros((RPS, HID), jnp.float32)
  b1 = enc_b1.reshape(1, HID)
  b2 = enc_b2.reshape(1, HID)
  ab1 = attr_b1.reshape(1, HID)
  ab2 = attr_b2.reshape(1, IN_DIM)
  sb1 = struct_b1.reshape(1, IN_DIM)

  degp = _make_propagate(16)(src_p, dst_p, ones16, z16)

  g1, dv = _stage1(x_p, enc_W1, degp)
  p1 = _make_propagate(HID)(src_p, dst_p, g1, z64)
  g2 = _stage2(p1, g1, dv, b1, enc_W2)
  p2 = _make_propagate(HID)(src_p, dst_p, g2, z64)
  g3a, g3b = _stage3(p2, g2, dv, b2, attr_W1)
  p3a = _make_propagate(HID)(src_p, dst_p, g3a, z64)
  p3b = _make_propagate(HID)(src_p, dst_p, g3b, z64)
  g4, h_ = _stage4(p3a, p3b, g3a, g3b, dv, ab1, struct_W1, sb1)
  p4 = _make_propagate(HID)(src_p, dst_p, g4, z64)
  x_ = _stage5(p4, g4, dv, attr_W2, ab2)

  adj = _adj(h_)
  return (x_[:N], adj)


# speedup vs baseline: 21.3203x; 1.0680x over previous
import functools

import jax
import jax.numpy as jnp
from jax import lax
from jax.experimental import pallas as pl
from jax.experimental.pallas import tpu as pltpu
from jax.experimental.pallas import tpu_sc as plsc

N = 10000
NP = 10240
E = 320000
IN_DIM = 128
HID = 64

NSC = 2
NSUB = 16
NW = NSC * NSUB
CH = 128
CPW = 8 * -(-E // (NW * CH * 8))
EP = NW * CH * CPW
RPS = NP // NSUB

_DUMMY_DST = N


@functools.lru_cache(maxsize=None)
def _make_propagate(D):
  mesh = plsc.VectorSubcoreMesh(core_axis_name="c", subcore_axis_name="s")
  nbuf = 8

  @functools.partial(
      pl.kernel,
      out_type=jax.ShapeDtypeStruct((NSC, NP, D), jnp.float32),
      mesh=mesh,
      compiler_params=pltpu.CompilerParams(use_tc_tiling_on_sc=False),
      scratch_types=[
          pltpu.VMEM((CPW, CH), jnp.int32),
          pltpu.VMEM((CPW, CH), jnp.int32),
          pltpu.VMEM((nbuf, CH, D), jnp.float32),
          pltpu.VMEM_SHARED((NP, D), jnp.float32),
          pltpu.SemaphoreType.DMA,
          pltpu.SemaphoreType.DMA((nbuf,)),
          pltpu.SemaphoreType.DMA,
          pltpu.SemaphoreType.DMA,
      ],
  )
  def prop(src_hbm, dst_hbm, g_hbm, zeros_hbm, out_hbm,
           src_v, dst_v, rows_v, acc, isem, gsems, ssemA, ssemB):
    c = lax.axis_index("c")
    s = lax.axis_index("s")
    w = s * NSC + c

    pltpu.sync_copy(zeros_hbm, acc.at[pl.ds(s * RPS, RPS)])
    pltpu.async_copy(src_hbm.at[pl.ds(w * CPW, CPW)], src_v, isem).wait()
    pltpu.async_copy(dst_hbm.at[pl.ds(w * CPW, CPW)], dst_v, isem).wait()
    plsc.subcore_barrier()

    half = nbuf // 2

    def fire_gather(j, b):
      pltpu.async_copy(g_hbm.at[src_v.at[j]], rows_v.at[b], gsems.at[b])

    def wait_gather(j, b):
      pltpu.make_async_copy(g_hbm.at[src_v.at[j]], rows_v.at[b],
                            gsems.at[b]).wait()

    def fire_scatter(j, b, sem):
      pltpu.async_copy(rows_v.at[b], acc.at[dst_v.at[j]], sem, add=True)

    def wait_scatter(j, b, sem):
      pltpu.make_async_copy(rows_v.at[b], acc.at[dst_v.at[j]], sem).wait()

    for b in range(half):
      fire_gather(b, b)

    @pl.loop(0, CPW, step=nbuf)
    def _(j):
      for b in range(half):
        wait_gather(j + b, b)
        fire_scatter(j + b, b, ssemA)
      @pl.when(j > 0)
      def _():
        for b in range(half):
          wait_scatter(j - half + b, half + b, ssemB)
      for b in range(half):
        fire_gather(j + half + b, half + b)
      for b in range(half):
        wait_gather(j + half + b, half + b)
        fire_scatter(j + half + b, half + b, ssemB)
      for b in range(half):
        wait_scatter(j + b, b, ssemA)
      @pl.when(j + nbuf < CPW)
      def _():
        for b in range(half):
          fire_gather(j + nbuf + b, b)

    for b in range(half):
      wait_scatter(CPW - half + b, half + b, ssemB)

    plsc.subcore_barrier()
    pltpu.sync_copy(acc.at[pl.ds(s * RPS, RPS)],
                    out_hbm.at[c, pl.ds(s * RPS, RPS)])

  return prop


DEGD = 16


def _make_degree():
  mesh = plsc.VectorSubcoreMesh(core_axis_name="c", subcore_axis_name="s")

  @functools.partial(
      pl.kernel,
      out_type=jax.ShapeDtypeStruct((NSC, NP, DEGD), jnp.float32),
      mesh=mesh,
      compiler_params=pltpu.CompilerParams(use_tc_tiling_on_sc=False),
      scratch_types=[
          pltpu.VMEM((CPW, CH), jnp.int32),
          pltpu.VMEM((CH, DEGD), jnp.float32),
          pltpu.VMEM_SHARED((NP, DEGD), jnp.float32),
          pltpu.SemaphoreType.DMA,
          pltpu.SemaphoreType.DMA,
      ],
  )
  def deg(dst_hbm, ones_hbm, zeros_hbm, out_hbm, dst_v, ones_v, acc,
          isem, ssem):
    c = lax.axis_index("c")
    s = lax.axis_index("s")
    w = s * NSC + c

    pltpu.sync_copy(zeros_hbm, acc.at[pl.ds(s * (NP // NSUB), NP // NSUB)])
    pltpu.sync_copy(ones_hbm, ones_v)
    pltpu.async_copy(dst_hbm.at[pl.ds(w * CPW, CPW)], dst_v, isem).wait()
    plsc.subcore_barrier()

    @pl.loop(0, CPW)
    def _(j):
      pltpu.async_copy(ones_v, acc.at[dst_v.at[j]], ssem, add=True)

    @pl.loop(0, CPW)
    def _(j):
      pltpu.make_async_copy(ones_v, acc.at[dst_v.at[j]], ssem).wait()

    plsc.subcore_barrier()
    pltpu.sync_copy(acc.at[pl.ds(s * (NP // NSUB), NP // NSUB)],
                    out_hbm.at[c, pl.ds(s * (NP // NSUB), NP // NSUB)])

  return deg


B = 512
GRID = (NP // B,)


def _row_spec(d):
  return pl.BlockSpec((B, d), lambda i: (i, 0))


def _part_spec(d):
  return pl.BlockSpec((NSC, B, d), lambda i: (0, i, 0))


def _full_spec(a, b):
  return pl.BlockSpec((a, b), lambda i: (0, 0))


def _stage1(x, w1, degp):
  def body(x_ref, w_ref, dp_ref, g1_ref, dv_ref):
    dp = dp_ref[0] + dp_ref[1]
    dinv = lax.rsqrt(1.0 + dp[:, 0:1])
    xw = jnp.dot(x_ref[...], w_ref[...], preferred_element_type=jnp.float32)
    g1_ref[...] = dinv * xw
    dv_ref[...] = jnp.broadcast_to(dinv, (B, 8))

  return pl.pallas_call(
      body,
      grid=GRID,
      in_specs=[_row_spec(IN_DIM), _full_spec(IN_DIM, HID), _part_spec(16)],
      out_specs=[_row_spec(HID), _row_spec(8)],
      out_shape=[jax.ShapeDtypeStruct((NP, HID), jnp.float32),
                 jax.ShapeDtypeStruct((NP, 8), jnp.float32)],
  )(x, w1, degp)


def _stage2(p1, g1, dv, b1, w2):
  def body(p_ref, g_ref, dv_ref, b_ref, w_ref, g2_ref):
    dinv = dv_ref[:, 0:1]
    h1 = jnp.maximum(dinv * (p_ref[0] + p_ref[1] + g_ref[...]) + b_ref[...],
                     0.0)
    g2_ref[...] = dinv * jnp.dot(h1, w_ref[...],
                                 preferred_element_type=jnp.float32)

  return pl.pallas_call(
      body,
      grid=GRID,
      in_specs=[_part_spec(HID), _row_spec(HID), _row_spec(8),
                _full_spec(1, HID), _full_spec(HID, HID)],
      out_specs=_row_spec(HID),
      out_shape=jax.ShapeDtypeStruct((NP, HID), jnp.float32),
  )(p1, g1, dv, b1, w2)


def _stage3(p2, g2, dv, b2, aw1):
  def body(p_ref, g_ref, dv_ref, b_ref, w_ref, g3a_ref, g3b_ref):
    dinv = dv_ref[:, 0:1]
    h = dinv * (p_ref[0] + p_ref[1] + g_ref[...]) + b_ref[...]
    hw = jnp.dot(h, w_ref[...], preferred_element_type=jnp.float32)
    g3a_ref[...] = dinv * hw
    g3b_ref[...] = dinv * h

  return pl.pallas_call(
      body,
      grid=GRID,
      in_specs=[_part_spec(HID), _row_spec(HID), _row_spec(8),
                _full_spec(1, HID), _full_spec(HID, HID)],
      out_specs=[_row_spec(HID), _row_spec(HID)],
      out_shape=[jax.ShapeDtypeStruct((NP, HID), jnp.float32),
                 jax.ShapeDtypeStruct((NP, HID), jnp.float32)],
  )(p2, g2, dv, b2, aw1)


def _stage4(p3a, p3b, g3a, g3b, dv, ab1, sw1, sb1):
  def body(pa_ref, pb_ref, ga_ref, gb_ref, dv_ref, ab_ref, sw_ref, sb_ref,
           g4_ref, h_ref):
    dinv = dv_ref[:, 0:1]
    pa = pa_ref[0] + pa_ref[1]
    x1 = jnp.maximum(dinv * (pa + ga_ref[...]) + ab_ref[...], 0.0)
    g4_ref[...] = dinv * x1
    pb = pb_ref[0] + pb_ref[1]
    hpre = dinv * (pb + gb_ref[...])
    h_ref[...] = jnp.dot(hpre, sw_ref[...],
                         preferred_element_type=jnp.float32) + sb_ref[...]

  return pl.pallas_call(
      body,
      grid=GRID,
      in_specs=[_part_spec(HID), _part_spec(HID), _row_spec(HID),
                _row_spec(HID), _row_spec(8),
                _full_spec(1, HID), _full_spec(HID, IN_DIM),
                _full_spec(1, IN_DIM)],
      out_specs=[_row_spec(HID), _row_spec(IN_DIM)],
      out_shape=[jax.ShapeDtypeStruct((NP, HID), jnp.float32),
                 jax.ShapeDtypeStruct((NP, IN_DIM), jnp.float32)],
  )(p3a, p3b, g3a, g3b, dv, ab1, sw1, sb1)


def _stage5(p4, g4, dv, aw2, ab2):
  def body(p_ref, g_ref, dv_ref, w_ref, b_ref, x_ref):
    dinv = dv_ref[:, 0:1]
    xpre = dinv * (p_ref[0] + p_ref[1] + g_ref[...])
    x_ref[...] = jnp.dot(xpre, w_ref[...],
                         preferred_element_type=jnp.float32) + b_ref[...]

  return pl.pallas_call(
      body,
      grid=GRID,
      in_specs=[_part_spec(HID), _row_spec(HID), _row_spec(8),
                _full_spec(HID, IN_DIM), _full_spec(1, IN_DIM)],
      out_specs=_row_spec(IN_DIM),
      out_shape=jax.ShapeDtypeStruct((NP, IN_DIM), jnp.float32),
  )(p4, g4, dv, aw2, ab2)


def _adj(h):
  BM, BN = 512, 1024

  def body(a_ref, b_ref, o_ref):
    o_ref[...] = lax.dot_general(
        a_ref[...], b_ref[...], (((1,), (1,)), ((), ())),
        preferred_element_type=jnp.float32)

  return pl.pallas_call(
      body,
      grid=(-(-N // BM), -(-N // BN)),
      in_specs=[pl.BlockSpec((BM, IN_DIM), lambda i, j: (i, 0)),
                pl.BlockSpec((BN, IN_DIM), lambda i, j: (j, 0))],
      out_specs=pl.BlockSpec((BM, BN), lambda i, j: (i, j)),
      out_shape=jax.ShapeDtypeStruct((N, N), jnp.float32),
  )(h, h)


def kernel(x, edge_index, enc_W1, enc_b1, enc_W2, enc_b2,
           attr_W1, attr_b1, attr_W2, attr_b2, struct_W1, struct_b1):
  src = edge_index[0]
  dst = edge_index[1]
  pad = EP - E
  spread = jnp.arange(pad, dtype=jnp.int32)
  src_p = jnp.concatenate([src, spread % N])
  src_p = src_p.reshape(NW * CPW, CH)
  dst_p = jnp.concatenate([dst, _DUMMY_DST + spread % (NP - N)])
  dst_p = dst_p.reshape(NW * CPW, CH)

  x_p = jnp.pad(x, ((0, NP - N), (0, 0)))
  ones16 = jnp.ones((CH, DEGD), jnp.float32)
  z16 = jnp.zeros((RPS, DEGD), jnp.float32)
  z64 = jnp.zeros((RPS, HID), jnp.float32)
  b1 = enc_b1.reshape(1, HID)
  b2 = enc_b2.reshape(1, HID)
  ab1 = attr_b1.reshape(1, HID)
  ab2 = attr_b2.reshape(1, IN_DIM)
  sb1 = struct_b1.reshape(1, IN_DIM)

  degp = _make_degree()(dst_p, ones16, z16)

  g1, dv = _stage1(x_p, enc_W1, degp)
  p1 = _make_propagate(HID)(src_p, dst_p, g1, z64)
  g2 = _stage2(p1, g1, dv, b1, enc_W2)
  p2 = _make_propagate(HID)(src_p, dst_p, g2, z64)
  g3a, g3b = _stage3(p2, g2, dv, b2, attr_W1)
  p3a = _make_propagate(HID)(src_p, dst_p, g3a, z64)
  p3b = _make_propagate(HID)(src_p, dst_p, g3b, z64)
  g4, h_ = _stage4(p3a, p3b, g3a, g3b, dv, ab1, struct_W1, sb1)
  p4 = _make_propagate(HID)(src_p, dst_p, g4, z64)
  x_ = _stage5(p4, g4, dv, attr_W2, ab2)

  adj = _adj(h_)
  return (x_[:N], adj)


# speedup vs baseline: 21.3309x; 1.0005x over previous
import functools

import jax
import jax.numpy as jnp
from jax import lax
from jax.experimental import pallas as pl
from jax.experimental.pallas import tpu as pltpu
from jax.experimental.pallas import tpu_sc as plsc

N = 10000
NP = 10240
E = 320000
IN_DIM = 128
HID = 64

NSC = 2
NSUB = 16
NW = NSC * NSUB
CH = 128
CPW = 8 * -(-E // (NW * CH * 8))
EP = NW * CH * CPW
RPS = NP // NSUB

_DUMMY_DST = N


@functools.lru_cache(maxsize=None)
def _make_propagate(D):
  mesh = plsc.VectorSubcoreMesh(core_axis_name="c", subcore_axis_name="s")
  nbuf = 8

  @functools.partial(
      pl.kernel,
      out_type=jax.ShapeDtypeStruct((NSC, NP, D), jnp.float32),
      mesh=mesh,
      compiler_params=pltpu.CompilerParams(use_tc_tiling_on_sc=False),
      scratch_types=[
          pltpu.VMEM((CPW, CH), jnp.int32),
          pltpu.VMEM((CPW, CH), jnp.int32),
          pltpu.VMEM((nbuf, CH, D), jnp.float32),
          pltpu.VMEM_SHARED((NP, D), jnp.float32),
          pltpu.SemaphoreType.DMA,
          pltpu.SemaphoreType.DMA((nbuf,)),
          pltpu.SemaphoreType.DMA,
          pltpu.SemaphoreType.DMA,
      ],
  )
  def prop(src_hbm, dst_hbm, g_hbm, zeros_hbm, out_hbm,
           src_v, dst_v, rows_v, acc, isem, gsems, ssemA, ssemB):
    c = lax.axis_index("c")
    s = lax.axis_index("s")
    w = s * NSC + c

    pltpu.sync_copy(zeros_hbm, acc.at[pl.ds(s * RPS, RPS)])
    pltpu.async_copy(src_hbm.at[pl.ds(w * CPW, CPW)], src_v, isem).wait()
    pltpu.async_copy(dst_hbm.at[pl.ds(w * CPW, CPW)], dst_v, isem).wait()
    plsc.subcore_barrier()

    half = nbuf // 2

    def fire_gather(j, b):
      pltpu.async_copy(g_hbm.at[src_v.at[j]], rows_v.at[b], gsems.at[b])

    def wait_gather(j, b):
      pltpu.make_async_copy(g_hbm.at[src_v.at[j]], rows_v.at[b],
                            gsems.at[b]).wait()

    def fire_scatter(j, b, sem):
      pltpu.async_copy(rows_v.at[b], acc.at[dst_v.at[j]], sem, add=True)

    def wait_scatter(j, b, sem):
      pltpu.make_async_copy(rows_v.at[b], acc.at[dst_v.at[j]], sem).wait()

    for b in range(half):
      fire_gather(b, b)

    @pl.loop(0, CPW, step=nbuf)
    def _(j):
      for b in range(half):
        wait_gather(j + b, b)
        fire_scatter(j + b, b, ssemA)
      @pl.when(j > 0)
      def _():
        for b in range(half):
          wait_scatter(j - half + b, half + b, ssemB)
      for b in range(half):
        fire_gather(j + half + b, half + b)
      for b in range(half):
        wait_gather(j + half + b, half + b)
        fire_scatter(j + half + b, half + b, ssemB)
      for b in range(half):
        wait_scatter(j + b, b, ssemA)
      @pl.when(j + nbuf < CPW)
      def _():
        for b in range(half):
          fire_gather(j + nbuf + b, b)

    for b in range(half):
      wait_scatter(CPW - half + b, half + b, ssemB)

    plsc.subcore_barrier()
    pltpu.sync_copy(acc.at[pl.ds(s * RPS, RPS)],
                    out_hbm.at[c, pl.ds(s * RPS, RPS)])

  return prop


DEGD = 16


def _make_degree():
  mesh = plsc.VectorSubcoreMesh(core_axis_name="c", subcore_axis_name="s")

  @functools.partial(
      pl.kernel,
      out_type=jax.ShapeDtypeStruct((NSC, NP, DEGD), jnp.float32),
      mesh=mesh,
      compiler_params=pltpu.CompilerParams(use_tc_tiling_on_sc=False),
      scratch_types=[
          pltpu.VMEM((CPW, CH), jnp.int32),
          pltpu.VMEM((CH, DEGD), jnp.float32),
          pltpu.VMEM_SHARED((NP, DEGD), jnp.float32),
          pltpu.SemaphoreType.DMA,
          pltpu.SemaphoreType.DMA,
      ],
  )
  def deg(dst_hbm, ones_hbm, zeros_hbm, out_hbm, dst_v, ones_v, acc,
          isem, ssem):
    c = lax.axis_index("c")
    s = lax.axis_index("s")
    w = s * NSC + c

    pltpu.sync_copy(zeros_hbm, acc.at[pl.ds(s * (NP // NSUB), NP // NSUB)])
    pltpu.sync_copy(ones_hbm, ones_v)
    pltpu.async_copy(dst_hbm.at[pl.ds(w * CPW, CPW)], dst_v, isem).wait()
    plsc.subcore_barrier()

    @pl.loop(0, CPW)
    def _(j):
      pltpu.async_copy(ones_v, acc.at[dst_v.at[j]], ssem, add=True)

    @pl.loop(0, CPW)
    def _(j):
      pltpu.make_async_copy(ones_v, acc.at[dst_v.at[j]], ssem).wait()

    plsc.subcore_barrier()
    pltpu.sync_copy(acc.at[pl.ds(s * (NP // NSUB), NP // NSUB)],
                    out_hbm.at[c, pl.ds(s * (NP // NSUB), NP // NSUB)])

  return deg


B = 512
GRID = (NP // B,)


def _row_spec(d):
  return pl.BlockSpec((B, d), lambda i: (i, 0))


def _part_spec(d):
  return pl.BlockSpec((NSC, B, d), lambda i: (0, i, 0))


def _full_spec(a, b):
  return pl.BlockSpec((a, b), lambda i: (0, 0))


def _stage1(x, w1, degp):
  def body(x_ref, w_ref, dp_ref, g1_ref, dv_ref):
    dp = dp_ref[0] + dp_ref[1]
    dinv = lax.rsqrt(1.0 + dp[:, 0:1])
    xw = jnp.dot(x_ref[...], w_ref[...], preferred_element_type=jnp.float32)
    g1_ref[...] = dinv * xw
    dv_ref[...] = jnp.broadcast_to(dinv, (B, 8))

  return pl.pallas_call(
      body,
      grid=GRID,
      in_specs=[_row_spec(IN_DIM), _full_spec(IN_DIM, HID), _part_spec(16)],
      out_specs=[_row_spec(HID), _row_spec(8)],
      out_shape=[jax.ShapeDtypeStruct((NP, HID), jnp.float32),
                 jax.ShapeDtypeStruct((NP, 8), jnp.float32)],
  )(x, w1, degp)


def _stage2(p1, g1, dv, b1, w2):
  def body(p_ref, g_ref, dv_ref, b_ref, w_ref, g2_ref):
    dinv = dv_ref[:, 0:1]
    h1 = jnp.maximum(dinv * (p_ref[0] + p_ref[1] + g_ref[...]) + b_ref[...],
                     0.0)
    g2_ref[...] = dinv * jnp.dot(h1, w_ref[...],
                                 preferred_element_type=jnp.float32)

  return pl.pallas_call(
      body,
      grid=GRID,
      in_specs=[_part_spec(HID), _row_spec(HID), _row_spec(8),
                _full_spec(1, HID), _full_spec(HID, HID)],
      out_specs=_row_spec(HID),
      out_shape=jax.ShapeDtypeStruct((NP, HID), jnp.float32),
  )(p1, g1, dv, b1, w2)


def _stage3(p2, g2, dv, b2, aw1):
  def body(p_ref, g_ref, dv_ref, b_ref, w_ref, g3a_ref, g3b_ref):
    dinv = dv_ref[:, 0:1]
    h = dinv * (p_ref[0] + p_ref[1] + g_ref[...]) + b_ref[...]
    hw = jnp.dot(h, w_ref[...], preferred_element_type=jnp.float32)
    g3a_ref[...] = dinv * hw
    g3b_ref[...] = dinv * h

  return pl.pallas_call(
      body,
      grid=GRID,
      in_specs=[_part_spec(HID), _row_spec(HID), _row_spec(8),
                _full_spec(1, HID), _full_spec(HID, HID)],
      out_specs=[_row_spec(HID), _row_spec(HID)],
      out_shape=[jax.ShapeDtypeStruct((NP, HID), jnp.float32),
                 jax.ShapeDtypeStruct((NP, HID), jnp.float32)],
  )(p2, g2, dv, b2, aw1)


def _stage4(p3a, p3b, g3a, g3b, dv, ab1, sw1, sb1):
  def body(pa_ref, pb_ref, ga_ref, gb_ref, dv_ref, ab_ref, sw_ref, sb_ref,
           g4_ref, h_ref):
    dinv = dv_ref[:, 0:1]
    pa = pa_ref[0] + pa_ref[1]
    x1 = jnp.maximum(dinv * (pa + ga_ref[...]) + ab_ref[...], 0.0)
    g4_ref[...] = dinv * x1
    pb = pb_ref[0] + pb_ref[1]
    hpre = dinv * (pb + gb_ref[...])
    h_ref[...] = jnp.dot(hpre, sw_ref[...],
                         preferred_element_type=jnp.float32) + sb_ref[...]

  return pl.pallas_call(
      body,
      grid=GRID,
      in_specs=[_part_spec(HID), _part_spec(HID), _row_spec(HID),
                _row_spec(HID), _row_spec(8),
                _full_spec(1, HID), _full_spec(HID, IN_DIM),
                _full_spec(1, IN_DIM)],
      out_specs=[_row_spec(HID), _row_spec(IN_DIM)],
      out_shape=[jax.ShapeDtypeStruct((NP, HID), jnp.float32),
                 jax.ShapeDtypeStruct((NP, IN_DIM), jnp.float32)],
  )(p3a, p3b, g3a, g3b, dv, ab1, sw1, sb1)


def _stage5(p4, g4, dv, aw2, ab2):
  def body(p_ref, g_ref, dv_ref, w_ref, b_ref, x_ref):
    dinv = dv_ref[:, 0:1]
    xpre = dinv * (p_ref[0] + p_ref[1] + g_ref[...])
    x_ref[...] = jnp.dot(xpre, w_ref[...],
                         preferred_element_type=jnp.float32) + b_ref[...]

  return pl.pallas_call(
      body,
      grid=GRID,
      in_specs=[_part_spec(HID), _row_spec(HID), _row_spec(8),
                _full_spec(HID, IN_DIM), _full_spec(1, IN_DIM)],
      out_specs=_row_spec(IN_DIM),
      out_shape=jax.ShapeDtypeStruct((NP, IN_DIM), jnp.float32),
  )(p4, g4, dv, aw2, ab2)


def _adj(h):
  BM, BN = 512, 1024

  def body(a_ref, b_ref, o_ref):
    a = a_ref[...].astype(jnp.bfloat16)
    b = b_ref[...].astype(jnp.bfloat16)
    o_ref[...] = lax.dot_general(
        a, b, (((1,), (1,)), ((), ())),
        preferred_element_type=jnp.float32)

  return pl.pallas_call(
      body,
      grid=(-(-N // BM), -(-N // BN)),
      in_specs=[pl.BlockSpec((BM, IN_DIM), lambda i, j: (i, 0)),
                pl.BlockSpec((BN, IN_DIM), lambda i, j: (j, 0))],
      out_specs=pl.BlockSpec((BM, BN), lambda i, j: (i, j)),
      out_shape=jax.ShapeDtypeStruct((N, N), jnp.float32),
  )(h, h)


def kernel(x, edge_index, enc_W1, enc_b1, enc_W2, enc_b2,
           attr_W1, attr_b1, attr_W2, attr_b2, struct_W1, struct_b1):
  src = edge_index[0]
  dst = edge_index[1]
  pad = EP - E
  spread = jnp.arange(pad, dtype=jnp.int32)
  src_p = jnp.concatenate([src, spread % N])
  src_p = src_p.reshape(NW * CPW, CH)
  dst_p = jnp.concatenate([dst, _DUMMY_DST + spread % (NP - N)])
  dst_p = dst_p.reshape(NW * CPW, CH)

  x_p = jnp.pad(x, ((0, NP - N), (0, 0)))
  ones16 = jnp.ones((CH, DEGD), jnp.float32)
  z16 = jnp.zeros((RPS, DEGD), jnp.float32)
  z64 = jnp.zeros((RPS, HID), jnp.float32)
  b1 = enc_b1.reshape(1, HID)
  b2 = enc_b2.reshape(1, HID)
  ab1 = attr_b1.reshape(1, HID)
  ab2 = attr_b2.reshape(1, IN_DIM)
  sb1 = struct_b1.reshape(1, IN_DIM)

  degp = _make_degree()(dst_p, ones16, z16)

  g1, dv = _stage1(x_p, enc_W1, degp)
  p1 = _make_propagate(HID)(src_p, dst_p, g1, z64)
  g2 = _stage2(p1, g1, dv, b1, enc_W2)
  p2 = _make_propagate(HID)(src_p, dst_p, g2, z64)
  g3a, g3b = _stage3(p2, g2, dv, b2, attr_W1)
  p3a = _make_propagate(HID)(src_p, dst_p, g3a, z64)
  p3b = _make_propagate(HID)(src_p, dst_p, g3b, z64)
  g4, h_ = _stage4(p3a, p3b, g3a, g3b, dv, ab1, struct_W1, sb1)
  p4 = _make_propagate(HID)(src_p, dst_p, g4, z64)
  x_ = _stage5(p4, g4, dv, attr_W2, ab2)

  adj = _adj(h_)
  return (x_[:N], adj)


# speedup vs baseline: 22.8598x; 1.0717x over previous
import functools

import jax
import jax.numpy as jnp
from jax import lax
from jax.experimental import pallas as pl
from jax.experimental.pallas import tpu as pltpu
from jax.experimental.pallas import tpu_sc as plsc

N = 10000
NP = 10240
E = 320000
IN_DIM = 128
HID = 64

NSC = 2
NSUB = 16
NW = NSC * NSUB
CH = 128
CPW = 8 * -(-E // (NW * CH * 8))
EP = NW * CH * CPW
RPS = NP // NSUB

_DUMMY_DST = N


@functools.lru_cache(maxsize=None)
def _make_propagate(D):
  mesh = plsc.VectorSubcoreMesh(core_axis_name="c", subcore_axis_name="s")
  nbuf = 8

  @functools.partial(
      pl.kernel,
      out_type=jax.ShapeDtypeStruct((NSC, NP, D), jnp.float32),
      mesh=mesh,
      compiler_params=pltpu.CompilerParams(use_tc_tiling_on_sc=False),
      scratch_types=[
          pltpu.VMEM((CPW, CH), jnp.int32),
          pltpu.VMEM((CPW, CH), jnp.int32),
          pltpu.VMEM((nbuf, CH, D), jnp.float32),
          pltpu.VMEM_SHARED((NP, D), jnp.float32),
          pltpu.SemaphoreType.DMA,
          pltpu.SemaphoreType.DMA((nbuf,)),
          pltpu.SemaphoreType.DMA,
          pltpu.SemaphoreType.DMA,
      ],
  )
  def prop(src_hbm, dst_hbm, g_hbm, zeros_hbm, out_hbm,
           src_v, dst_v, rows_v, acc, isem, gsems, ssemA, ssemB):
    c = lax.axis_index("c")
    s = lax.axis_index("s")
    w = s * NSC + c

    pltpu.sync_copy(zeros_hbm, acc.at[pl.ds(s * RPS, RPS)])
    pltpu.async_copy(src_hbm.at[pl.ds(w * CPW, CPW)], src_v, isem).wait()
    pltpu.async_copy(dst_hbm.at[pl.ds(w * CPW, CPW)], dst_v, isem).wait()
    plsc.subcore_barrier()

    half = nbuf // 2

    def fire_gather(j, b):
      pltpu.async_copy(g_hbm.at[src_v.at[j]], rows_v.at[b], gsems.at[b])

    def wait_gather(j, b):
      pltpu.make_async_copy(g_hbm.at[src_v.at[j]], rows_v.at[b],
                            gsems.at[b]).wait()

    def fire_scatter(j, b, sem):
      pltpu.async_copy(rows_v.at[b], acc.at[dst_v.at[j]], sem, add=True)

    def wait_scatter(j, b, sem):
      pltpu.make_async_copy(rows_v.at[b], acc.at[dst_v.at[j]], sem).wait()

    for b in range(half):
      fire_gather(b, b)

    @pl.loop(0, CPW, step=nbuf)
    def _(j):
      for b in range(half):
        wait_gather(j + b, b)
        fire_scatter(j + b, b, ssemA)
      @pl.when(j > 0)
      def _():
        for b in range(half):
          wait_scatter(j - half + b, half + b, ssemB)
      for b in range(half):
        fire_gather(j + half + b, half + b)
      for b in range(half):
        wait_gather(j + half + b, half + b)
        fire_scatter(j + half + b, half + b, ssemB)
      for b in range(half):
        wait_scatter(j + b, b, ssemA)
      @pl.when(j + nbuf < CPW)
      def _():
        for b in range(half):
          fire_gather(j + nbuf + b, b)

    for b in range(half):
      wait_scatter(CPW - half + b, half + b, ssemB)

    plsc.subcore_barrier()
    pltpu.sync_copy(acc.at[pl.ds(s * RPS, RPS)],
                    out_hbm.at[c, pl.ds(s * RPS, RPS)])

  return prop


DEGD = 16


def _make_degree():
  mesh = plsc.VectorSubcoreMesh(core_axis_name="c", subcore_axis_name="s")

  @functools.partial(
      pl.kernel,
      out_type=jax.ShapeDtypeStruct((NSC, NP, DEGD), jnp.float32),
      mesh=mesh,
      compiler_params=pltpu.CompilerParams(use_tc_tiling_on_sc=False),
      scratch_types=[
          pltpu.VMEM((CPW, CH), jnp.int32),
          pltpu.VMEM((CH, DEGD), jnp.float32),
          pltpu.VMEM_SHARED((NP, DEGD), jnp.float32),
          pltpu.SemaphoreType.DMA,
          pltpu.SemaphoreType.DMA,
      ],
  )
  def deg(dst_hbm, ones_hbm, zeros_hbm, out_hbm, dst_v, ones_v, acc,
          isem, ssem):
    c = lax.axis_index("c")
    s = lax.axis_index("s")
    w = s * NSC + c

    pltpu.sync_copy(zeros_hbm, acc.at[pl.ds(s * (NP // NSUB), NP // NSUB)])
    pltpu.sync_copy(ones_hbm, ones_v)
    pltpu.async_copy(dst_hbm.at[pl.ds(w * CPW, CPW)], dst_v, isem).wait()
    plsc.subcore_barrier()

    @pl.loop(0, CPW)
    def _(j):
      pltpu.async_copy(ones_v, acc.at[dst_v.at[j]], ssem, add=True)

    @pl.loop(0, CPW)
    def _(j):
      pltpu.make_async_copy(ones_v, acc.at[dst_v.at[j]], ssem).wait()

    plsc.subcore_barrier()
    pltpu.sync_copy(acc.at[pl.ds(s * (NP // NSUB), NP // NSUB)],
                    out_hbm.at[c, pl.ds(s * (NP // NSUB), NP // NSUB)])

  return deg


B = 512
GRID = (NP // B,)


def _row_spec(d):
  return pl.BlockSpec((B, d), lambda i: (i, 0))


def _part_spec(d):
  return pl.BlockSpec((NSC, B, d), lambda i: (0, i, 0))


def _full_spec(a, b):
  return pl.BlockSpec((a, b), lambda i: (0, 0))


def _stage1(x, w1, degp):
  def body(x_ref, w_ref, dp_ref, g1_ref, dv_ref):
    dp = dp_ref[0] + dp_ref[1]
    dinv = lax.rsqrt(1.0 + dp[:, 0:1])
    xw = jnp.dot(x_ref[...], w_ref[...], preferred_element_type=jnp.float32)
    g1_ref[...] = dinv * xw
    dv_ref[...] = jnp.broadcast_to(dinv, (B, 8))

  return pl.pallas_call(
      body,
      grid=GRID,
      in_specs=[_row_spec(IN_DIM), _full_spec(IN_DIM, HID), _part_spec(16)],
      out_specs=[_row_spec(HID), _row_spec(8)],
      out_shape=[jax.ShapeDtypeStruct((NP, HID), jnp.float32),
                 jax.ShapeDtypeStruct((NP, 8), jnp.float32)],
  )(x, w1, degp)


def _stage2(p1, g1, dv, b1, w2):
  def body(p_ref, g_ref, dv_ref, b_ref, w_ref, g2_ref):
    dinv = dv_ref[:, 0:1]
    h1 = jnp.maximum(dinv * (p_ref[0] + p_ref[1] + g_ref[...]) + b_ref[...],
                     0.0)
    g2_ref[...] = dinv * jnp.dot(h1, w_ref[...],
                                 preferred_element_type=jnp.float32)

  return pl.pallas_call(
      body,
      grid=GRID,
      in_specs=[_part_spec(HID), _row_spec(HID), _row_spec(8),
                _full_spec(1, HID), _full_spec(HID, HID)],
      out_specs=_row_spec(HID),
      out_shape=jax.ShapeDtypeStruct((NP, HID), jnp.float32),
  )(p1, g1, dv, b1, w2)


def _stage3(p2, g2, dv, b2, aw1):
  def body(p_ref, g_ref, dv_ref, b_ref, w_ref, g3a_ref, g3b_ref):
    dinv = dv_ref[:, 0:1]
    h = dinv * (p_ref[0] + p_ref[1] + g_ref[...]) + b_ref[...]
    hw = jnp.dot(h, w_ref[...], preferred_element_type=jnp.float32)
    g3a_ref[...] = dinv * hw
    g3b_ref[...] = dinv * h

  return pl.pallas_call(
      body,
      grid=GRID,
      in_specs=[_part_spec(HID), _row_spec(HID), _row_spec(8),
                _full_spec(1, HID), _full_spec(HID, HID)],
      out_specs=[_row_spec(HID), _row_spec(HID)],
      out_shape=[jax.ShapeDtypeStruct((NP, HID), jnp.float32),
                 jax.ShapeDtypeStruct((NP, HID), jnp.float32)],
  )(p2, g2, dv, b2, aw1)


def _stage4(p3a, p3b, g3a, g3b, dv, ab1, sw1, sb1):
  def body(pa_ref, pb_ref, ga_ref, gb_ref, dv_ref, ab_ref, sw_ref, sb_ref,
           g4_ref, h_ref):
    dinv = dv_ref[:, 0:1]
    pa = pa_ref[0] + pa_ref[1]
    x1 = jnp.maximum(dinv * (pa + ga_ref[...]) + ab_ref[...], 0.0)
    g4_ref[...] = dinv * x1
    pb = pb_ref[0] + pb_ref[1]
    hpre = dinv * (pb + gb_ref[...])
    h_ref[...] = jnp.dot(hpre, sw_ref[...],
                         preferred_element_type=jnp.float32) + sb_ref[...]

  return pl.pallas_call(
      body,
      grid=GRID,
      in_specs=[_part_spec(HID), _part_spec(HID), _row_spec(HID),
                _row_spec(HID), _row_spec(8),
                _full_spec(1, HID), _full_spec(HID, IN_DIM),
                _full_spec(1, IN_DIM)],
      out_specs=[_row_spec(HID), _row_spec(IN_DIM)],
      out_shape=[jax.ShapeDtypeStruct((NP, HID), jnp.float32),
                 jax.ShapeDtypeStruct((NP, IN_DIM), jnp.float32)],
  )(p3a, p3b, g3a, g3b, dv, ab1, sw1, sb1)


def _stage5(p4, g4, dv, aw2, ab2):
  def body(p_ref, g_ref, dv_ref, w_ref, b_ref, x_ref):
    dinv = dv_ref[:, 0:1]
    xpre = dinv * (p_ref[0] + p_ref[1] + g_ref[...])
    x_ref[...] = jnp.dot(xpre, w_ref[...],
                         preferred_element_type=jnp.float32) + b_ref[...]

  return pl.pallas_call(
      body,
      grid=GRID,
      in_specs=[_part_spec(HID), _row_spec(HID), _row_spec(8),
                _full_spec(HID, IN_DIM), _full_spec(1, IN_DIM)],
      out_specs=_row_spec(IN_DIM),
      out_shape=jax.ShapeDtypeStruct((NP, IN_DIM), jnp.float32),
  )(p4, g4, dv, aw2, ab2)


def _adj(h):
  BM, BN = 512, 1024

  def body(a_ref, b_ref, o_ref):
    j = pl.program_id(1)
    a = a_ref[...].astype(jnp.bfloat16)
    b = b_ref[pl.ds(j * BN, BN), :].astype(jnp.bfloat16)
    o_ref[...] = lax.dot_general(
        a, b, (((1,), (1,)), ((), ())),
        preferred_element_type=jnp.float32)

  return pl.pallas_call(
      body,
      grid=(-(-N // BM), -(-N // BN)),
      in_specs=[pl.BlockSpec((BM, IN_DIM), lambda i, j: (i, 0)),
                pl.BlockSpec((NP, IN_DIM), lambda i, j: (0, 0))],
      out_specs=pl.BlockSpec((BM, BN), lambda i, j: (i, j)),
      out_shape=jax.ShapeDtypeStruct((N, N), jnp.float32),
  )(h, h)


def kernel(x, edge_index, enc_W1, enc_b1, enc_W2, enc_b2,
           attr_W1, attr_b1, attr_W2, attr_b2, struct_W1, struct_b1):
  src = edge_index[0]
  dst = edge_index[1]
  pad = EP - E
  spread = jnp.arange(pad, dtype=jnp.int32)
  src_p = jnp.concatenate([src, spread % N])
  src_p = src_p.reshape(NW * CPW, CH)
  dst_p = jnp.concatenate([dst, _DUMMY_DST + spread % (NP - N)])
  dst_p = dst_p.reshape(NW * CPW, CH)

  x_p = jnp.pad(x, ((0, NP - N), (0, 0)))
  ones16 = jnp.ones((CH, DEGD), jnp.float32)
  z16 = jnp.zeros((RPS, DEGD), jnp.float32)
  z64 = jnp.zeros((RPS, HID), jnp.float32)
  b1 = enc_b1.reshape(1, HID)
  b2 = enc_b2.reshape(1, HID)
  ab1 = attr_b1.reshape(1, HID)
  ab2 = attr_b2.reshape(1, IN_DIM)
  sb1 = struct_b1.reshape(1, IN_DIM)

  degp = _make_degree()(dst_p, ones16, z16)

  g1, dv = _stage1(x_p, enc_W1, degp)
  p1 = _make_propagate(HID)(src_p, dst_p, g1, z64)
  g2 = _stage2(p1, g1, dv, b1, enc_W2)
  p2 = _make_propagate(HID)(src_p, dst_p, g2, z64)
  g3a, g3b = _stage3(p2, g2, dv, b2, attr_W1)
  p3a = _make_propagate(HID)(src_p, dst_p, g3a, z64)
  p3b = _make_propagate(HID)(src_p, dst_p, g3b, z64)
  g4, h_ = _stage4(p3a, p3b, g3a, g3b, dv, ab1, struct_W1, sb1)
  p4 = _make_propagate(HID)(src_p, dst_p, g4, z64)
  x_ = _stage5(p4, g4, dv, attr_W2, ab2)

  adj = _adj(h_)
  return (x_[:N], adj)


# speedup vs baseline: 24.6350x; 1.0777x over previous
import functools

import jax
import jax.numpy as jnp
from jax import lax
from jax.experimental import pallas as pl
from jax.experimental.pallas import tpu as pltpu
from jax.experimental.pallas import tpu_sc as plsc

N = 10000
NP = 10240
E = 320000
IN_DIM = 128
HID = 64

NSC = 2
NSUB = 16
NW = NSC * NSUB
CH = 128
CPW = 8 * -(-E // (NW * CH * 8))
EP = NW * CH * CPW
RPS = NP // NSUB

_DUMMY_DST = N


@functools.lru_cache(maxsize=None)
def _make_propagate(D):
  mesh = plsc.VectorSubcoreMesh(core_axis_name="c", subcore_axis_name="s")
  nbuf = 8

  @functools.partial(
      pl.kernel,
      out_type=jax.ShapeDtypeStruct((NSC, NP, D), jnp.float32),
      mesh=mesh,
      compiler_params=pltpu.CompilerParams(use_tc_tiling_on_sc=False),
      scratch_types=[
          pltpu.VMEM((CPW, CH), jnp.int32),
          pltpu.VMEM((CPW, CH), jnp.int32),
          pltpu.VMEM((nbuf, CH, D), jnp.float32),
          pltpu.VMEM_SHARED((NP, D), jnp.float32),
          pltpu.SemaphoreType.DMA,
          pltpu.SemaphoreType.DMA((nbuf,)),
          pltpu.SemaphoreType.DMA,
          pltpu.SemaphoreType.DMA,
      ],
  )
  def prop(src_hbm, dst_hbm, g_hbm, zeros_hbm, out_hbm,
           src_v, dst_v, rows_v, acc, isem, gsems, ssemA, ssemB):
    c = lax.axis_index("c")
    s = lax.axis_index("s")
    w = s * NSC + c

    pltpu.sync_copy(zeros_hbm, acc.at[pl.ds(s * RPS, RPS)])
    pltpu.async_copy(src_hbm.at[pl.ds(w * CPW, CPW)], src_v, isem).wait()
    pltpu.async_copy(dst_hbm.at[pl.ds(w * CPW, CPW)], dst_v, isem).wait()
    plsc.subcore_barrier()

    half = nbuf // 2

    def fire_gather(j, b):
      pltpu.async_copy(g_hbm.at[src_v.at[j]], rows_v.at[b], gsems.at[b])

    def wait_gather(j, b):
      pltpu.make_async_copy(g_hbm.at[src_v.at[j]], rows_v.at[b],
                            gsems.at[b]).wait()

    def fire_scatter(j, b, sem):
      pltpu.async_copy(rows_v.at[b], acc.at[dst_v.at[j]], sem, add=True)

    def wait_scatter(j, b, sem):
      pltpu.make_async_copy(rows_v.at[b], acc.at[dst_v.at[j]], sem).wait()

    for b in range(half):
      fire_gather(b, b)

    @pl.loop(0, CPW, step=nbuf)
    def _(j):
      for b in range(half):
        wait_gather(j + b, b)
        fire_scatter(j + b, b, ssemA)
      @pl.when(j > 0)
      def _():
        for b in range(half):
          wait_scatter(j - half + b, half + b, ssemB)
      for b in range(half):
        fire_gather(j + half + b, half + b)
      for b in range(half):
        wait_gather(j + half + b, half + b)
        fire_scatter(j + half + b, half + b, ssemB)
      for b in range(half):
        wait_scatter(j + b, b, ssemA)
      @pl.when(j + nbuf < CPW)
      def _():
        for b in range(half):
          fire_gather(j + nbuf + b, b)

    for b in range(half):
      wait_scatter(CPW - half + b, half + b, ssemB)

    plsc.subcore_barrier()
    pltpu.sync_copy(acc.at[pl.ds(s * RPS, RPS)],
                    out_hbm.at[c, pl.ds(s * RPS, RPS)])

  return prop


DEGD = 16


def _make_degree():
  mesh = plsc.VectorSubcoreMesh(core_axis_name="c", subcore_axis_name="s")

  @functools.partial(
      pl.kernel,
      out_type=jax.ShapeDtypeStruct((NSC, NP, DEGD), jnp.float32),
      mesh=mesh,
      compiler_params=pltpu.CompilerParams(use_tc_tiling_on_sc=False),
      scratch_types=[
          pltpu.VMEM((CPW, CH), jnp.int32),
          pltpu.VMEM((CH, DEGD), jnp.float32),
          pltpu.VMEM_SHARED((NP, DEGD), jnp.float32),
          pltpu.SemaphoreType.DMA,
          pltpu.SemaphoreType.DMA,
      ],
  )
  def deg(dst_hbm, ones_hbm, zeros_hbm, out_hbm, dst_v, ones_v, acc,
          isem, ssem):
    c = lax.axis_index("c")
    s = lax.axis_index("s")
    w = s * NSC + c

    pltpu.sync_copy(zeros_hbm, acc.at[pl.ds(s * (NP // NSUB), NP // NSUB)])
    pltpu.sync_copy(ones_hbm, ones_v)
    pltpu.async_copy(dst_hbm.at[pl.ds(w * CPW, CPW)], dst_v, isem).wait()
    plsc.subcore_barrier()

    @pl.loop(0, CPW)
    def _(j):
      pltpu.async_copy(ones_v, acc.at[dst_v.at[j]], ssem, add=True)

    @pl.loop(0, CPW)
    def _(j):
      pltpu.make_async_copy(ones_v, acc.at[dst_v.at[j]], ssem).wait()

    plsc.subcore_barrier()
    pltpu.sync_copy(acc.at[pl.ds(s * (NP // NSUB), NP // NSUB)],
                    out_hbm.at[c, pl.ds(s * (NP // NSUB), NP // NSUB)])

  return deg


B = 512
GRID = (NP // B,)


def _row_spec(d):
  return pl.BlockSpec((B, d), lambda i: (i, 0))


def _part_spec(d):
  return pl.BlockSpec((NSC, B, d), lambda i: (0, i, 0))


def _full_spec(a, b):
  return pl.BlockSpec((a, b), lambda i: (0, 0))


def _stage1(x, w1, degp):
  def body(x_ref, w_ref, dp_ref, g1_ref, dv_ref):
    dp = dp_ref[0] + dp_ref[1]
    dinv = lax.rsqrt(1.0 + dp[:, 0:1])
    xw = jnp.dot(x_ref[...], w_ref[...], preferred_element_type=jnp.float32)
    g1_ref[...] = dinv * xw
    dv_ref[...] = jnp.broadcast_to(dinv, (B, 8))

  return pl.pallas_call(
      body,
      grid=GRID,
      in_specs=[_row_spec(IN_DIM), _full_spec(IN_DIM, HID), _part_spec(16)],
      out_specs=[_row_spec(HID), _row_spec(8)],
      out_shape=[jax.ShapeDtypeStruct((NP, HID), jnp.float32),
                 jax.ShapeDtypeStruct((NP, 8), jnp.float32)],
  )(x, w1, degp)


def _stage2(p1, g1, dv, b1, w2):
  def body(p_ref, g_ref, dv_ref, b_ref, w_ref, g2_ref):
    dinv = dv_ref[:, 0:1]
    h1 = jnp.maximum(dinv * (p_ref[0] + p_ref[1] + g_ref[...]) + b_ref[...],
                     0.0)
    g2_ref[...] = dinv * jnp.dot(h1, w_ref[...],
                                 preferred_element_type=jnp.float32)

  return pl.pallas_call(
      body,
      grid=GRID,
      in_specs=[_part_spec(HID), _row_spec(HID), _row_spec(8),
                _full_spec(1, HID), _full_spec(HID, HID)],
      out_specs=_row_spec(HID),
      out_shape=jax.ShapeDtypeStruct((NP, HID), jnp.float32),
  )(p1, g1, dv, b1, w2)


def _stage3(p2, g2, dv, b2, aw1):
  def body(p_ref, g_ref, dv_ref, b_ref, w_ref, g3a_ref, g3b_ref):
    dinv = dv_ref[:, 0:1]
    h = dinv * (p_ref[0] + p_ref[1] + g_ref[...]) + b_ref[...]
    hw = jnp.dot(h, w_ref[...], preferred_element_type=jnp.float32)
    g3a_ref[...] = dinv * hw
    g3b_ref[...] = dinv * h

  return pl.pallas_call(
      body,
      grid=GRID,
      in_specs=[_part_spec(HID), _row_spec(HID), _row_spec(8),
                _full_spec(1, HID), _full_spec(HID, HID)],
      out_specs=[_row_spec(HID), _row_spec(HID)],
      out_shape=[jax.ShapeDtypeStruct((NP, HID), jnp.float32),
                 jax.ShapeDtypeStruct((NP, HID), jnp.float32)],
  )(p2, g2, dv, b2, aw1)


def _stage4(p3a, p3b, g3a, g3b, dv, ab1, sw1, sb1):
  def body(pa_ref, pb_ref, ga_ref, gb_ref, dv_ref, ab_ref, sw_ref, sb_ref,
           g4_ref, h_ref):
    dinv = dv_ref[:, 0:1]
    pa = pa_ref[0] + pa_ref[1]
    x1 = jnp.maximum(dinv * (pa + ga_ref[...]) + ab_ref[...], 0.0)
    g4_ref[...] = dinv * x1
    pb = pb_ref[0] + pb_ref[1]
    hpre = dinv * (pb + gb_ref[...])
    h_ref[...] = jnp.dot(hpre, sw_ref[...],
                         preferred_element_type=jnp.float32) + sb_ref[...]

  return pl.pallas_call(
      body,
      grid=GRID,
      in_specs=[_part_spec(HID), _part_spec(HID), _row_spec(HID),
                _row_spec(HID), _row_spec(8),
                _full_spec(1, HID), _full_spec(HID, IN_DIM),
                _full_spec(1, IN_DIM)],
      out_specs=[_row_spec(HID), _row_spec(IN_DIM)],
      out_shape=[jax.ShapeDtypeStruct((NP, HID), jnp.float32),
                 jax.ShapeDtypeStruct((NP, IN_DIM), jnp.float32)],
  )(p3a, p3b, g3a, g3b, dv, ab1, sw1, sb1)


def _stage5(p4, g4, dv, aw2, ab2):
  def body(p_ref, g_ref, dv_ref, w_ref, b_ref, x_ref):
    dinv = dv_ref[:, 0:1]
    xpre = dinv * (p_ref[0] + p_ref[1] + g_ref[...])
    x_ref[...] = jnp.dot(xpre, w_ref[...],
                         preferred_element_type=jnp.float32) + b_ref[...]

  return pl.pallas_call(
      body,
      grid=GRID,
      in_specs=[_part_spec(HID), _row_spec(HID), _row_spec(8),
                _full_spec(HID, IN_DIM), _full_spec(1, IN_DIM)],
      out_specs=_row_spec(IN_DIM),
      out_shape=jax.ShapeDtypeStruct((NP, IN_DIM), jnp.float32),
  )(p4, g4, dv, aw2, ab2)


def _adj(h):
  BM = 512

  def body(a_ref, b_ref, o_ref):
    a = a_ref[...].astype(jnp.bfloat16)
    b = b_ref[:N, :].astype(jnp.bfloat16)
    o_ref[...] = lax.dot_general(
        a, b, (((1,), (1,)), ((), ())),
        preferred_element_type=jnp.float32)

  return pl.pallas_call(
      body,
      grid=(-(-N // BM),),
      in_specs=[pl.BlockSpec((BM, IN_DIM), lambda i: (i, 0)),
                pl.BlockSpec((NP, IN_DIM), lambda i: (0, 0))],
      out_specs=pl.BlockSpec((BM, N), lambda i: (i, 0)),
      out_shape=jax.ShapeDtypeStruct((N, N), jnp.float32),
  )(h, h)


def kernel(x, edge_index, enc_W1, enc_b1, enc_W2, enc_b2,
           attr_W1, attr_b1, attr_W2, attr_b2, struct_W1, struct_b1):
  src = edge_index[0]
  dst = edge_index[1]
  pad = EP - E
  spread = jnp.arange(pad, dtype=jnp.int32)
  src_p = jnp.concatenate([src, spread % N])
  src_p = src_p.reshape(NW * CPW, CH)
  dst_p = jnp.concatenate([dst, _DUMMY_DST + spread % (NP - N)])
  dst_p = dst_p.reshape(NW * CPW, CH)

  x_p = jnp.pad(x, ((0, NP - N), (0, 0)))
  ones16 = jnp.ones((CH, DEGD), jnp.float32)
  z16 = jnp.zeros((RPS, DEGD), jnp.float32)
  z64 = jnp.zeros((RPS, HID), jnp.float32)
  b1 = enc_b1.reshape(1, HID)
  b2 = enc_b2.reshape(1, HID)
  ab1 = attr_b1.reshape(1, HID)
  ab2 = attr_b2.reshape(1, IN_DIM)
  sb1 = struct_b1.reshape(1, IN_DIM)

  degp = _make_degree()(dst_p, ones16, z16)

  g1, dv = _stage1(x_p, enc_W1, degp)
  p1 = _make_propagate(HID)(src_p, dst_p, g1, z64)
  g2 = _stage2(p1, g1, dv, b1, enc_W2)
  p2 = _make_propagate(HID)(src_p, dst_p, g2, z64)
  g3a, g3b = _stage3(p2, g2, dv, b2, attr_W1)
  p3a = _make_propagate(HID)(src_p, dst_p, g3a, z64)
  p3b = _make_propagate(HID)(src_p, dst_p, g3b, z64)
  g4, h_ = _stage4(p3a, p3b, g3a, g3b, dv, ab1, struct_W1, sb1)
  p4 = _make_propagate(HID)(src_p, dst_p, g4, z64)
  x_ = _stage5(p4, g4, dv, attr_W2, ab2)

  adj = _adj(h_)
  return (x_[:N], adj)


# speedup vs baseline: 25.3572x; 1.0293x over previous
import functools

import jax
import jax.numpy as jnp
from jax import lax
from jax.experimental import pallas as pl
from jax.experimental.pallas import tpu as pltpu
from jax.experimental.pallas import tpu_sc as plsc

N = 10000
NP = 10240
E = 320000
IN_DIM = 128
HID = 64

NSC = 2
NSUB = 16
NW = NSC * NSUB
CH = 128
CPW = 8 * -(-E // (NW * CH * 8))
EP = NW * CH * CPW
RPS = NP // NSUB

_DUMMY_DST = N


@functools.lru_cache(maxsize=None)
def _make_propagate(D):
  mesh = plsc.VectorSubcoreMesh(core_axis_name="c", subcore_axis_name="s")
  nbuf = 8

  @functools.partial(
      pl.kernel,
      out_type=jax.ShapeDtypeStruct((NSC, NP, D), jnp.float32),
      mesh=mesh,
      compiler_params=pltpu.CompilerParams(use_tc_tiling_on_sc=False),
      scratch_types=[
          pltpu.VMEM((CPW, CH), jnp.int32),
          pltpu.VMEM((CPW, CH), jnp.int32),
          pltpu.VMEM((nbuf, CH, D), jnp.float32),
          pltpu.VMEM_SHARED((NP, D), jnp.float32),
          pltpu.SemaphoreType.DMA,
          pltpu.SemaphoreType.DMA((nbuf,)),
          pltpu.SemaphoreType.DMA,
          pltpu.SemaphoreType.DMA,
      ],
  )
  def prop(src_hbm, dst_hbm, g_hbm, zeros_hbm, out_hbm,
           src_v, dst_v, rows_v, acc, isem, gsems, ssemA, ssemB):
    c = lax.axis_index("c")
    s = lax.axis_index("s")
    w = s * NSC + c

    pltpu.sync_copy(zeros_hbm, acc.at[pl.ds(s * RPS, RPS)])
    pltpu.async_copy(src_hbm.at[pl.ds(w * CPW, CPW)], src_v, isem).wait()
    pltpu.async_copy(dst_hbm.at[pl.ds(w * CPW, CPW)], dst_v, isem).wait()
    plsc.subcore_barrier()

    half = nbuf // 2

    def fire_gather(j, b):
      pltpu.async_copy(g_hbm.at[src_v.at[j]], rows_v.at[b], gsems.at[b])

    def wait_gather(j, b):
      pltpu.make_async_copy(g_hbm.at[src_v.at[j]], rows_v.at[b],
                            gsems.at[b]).wait()

    def fire_scatter(j, b, sem):
      pltpu.async_copy(rows_v.at[b], acc.at[dst_v.at[j]], sem, add=True)

    def wait_scatter(j, b, sem):
      pltpu.make_async_copy(rows_v.at[b], acc.at[dst_v.at[j]], sem).wait()

    for b in range(half):
      fire_gather(b, b)

    @pl.loop(0, CPW, step=nbuf)
    def _(j):
      for b in range(half):
        wait_gather(j + b, b)
        fire_scatter(j + b, b, ssemA)
      @pl.when(j > 0)
      def _():
        for b in range(half):
          wait_scatter(j - half + b, half + b, ssemB)
      for b in range(half):
        fire_gather(j + half + b, half + b)
      for b in range(half):
        wait_gather(j + half + b, half + b)
        fire_scatter(j + half + b, half + b, ssemB)
      for b in range(half):
        wait_scatter(j + b, b, ssemA)
      @pl.when(j + nbuf < CPW)
      def _():
        for b in range(half):
          fire_gather(j + nbuf + b, b)

    for b in range(half):
      wait_scatter(CPW - half + b, half + b, ssemB)

    plsc.subcore_barrier()
    pltpu.sync_copy(acc.at[pl.ds(s * RPS, RPS)],
                    out_hbm.at[c, pl.ds(s * RPS, RPS)])

  return prop


DEGD = 16


def _make_degree():
  mesh = plsc.VectorSubcoreMesh(core_axis_name="c", subcore_axis_name="s")

  @functools.partial(
      pl.kernel,
      out_type=jax.ShapeDtypeStruct((NSC, NP, DEGD), jnp.float32),
      mesh=mesh,
      compiler_params=pltpu.CompilerParams(use_tc_tiling_on_sc=False),
      scratch_types=[
          pltpu.VMEM((CPW, CH), jnp.int32),
          pltpu.VMEM((CH, DEGD), jnp.float32),
          pltpu.VMEM_SHARED((NP, DEGD), jnp.float32),
          pltpu.SemaphoreType.DMA,
          pltpu.SemaphoreType.DMA,
      ],
  )
  def deg(dst_hbm, ones_hbm, zeros_hbm, out_hbm, dst_v, ones_v, acc,
          isem, ssem):
    c = lax.axis_index("c")
    s = lax.axis_index("s")
    w = s * NSC + c

    pltpu.sync_copy(zeros_hbm, acc.at[pl.ds(s * (NP // NSUB), NP // NSUB)])
    pltpu.sync_copy(ones_hbm, ones_v)
    pltpu.async_copy(dst_hbm.at[pl.ds(w * CPW, CPW)], dst_v, isem).wait()
    plsc.subcore_barrier()

    @pl.loop(0, CPW)
    def _(j):
      pltpu.async_copy(ones_v, acc.at[dst_v.at[j]], ssem, add=True)

    @pl.loop(0, CPW)
    def _(j):
      pltpu.make_async_copy(ones_v, acc.at[dst_v.at[j]], ssem).wait()

    plsc.subcore_barrier()
    pltpu.sync_copy(acc.at[pl.ds(s * (NP // NSUB), NP // NSUB)],
                    out_hbm.at[c, pl.ds(s * (NP // NSUB), NP // NSUB)])

  return deg


B = 512
GRID = (NP // B,)


def _row_spec(d):
  return pl.BlockSpec((B, d), lambda i: (i, 0))


def _part_spec(d):
  return pl.BlockSpec((NSC, B, d), lambda i: (0, i, 0))


def _full_spec(a, b):
  return pl.BlockSpec((a, b), lambda i: (0, 0))


def _stage1(x, w1, degp):
  def body(x_ref, w_ref, dp_ref, g1_ref, dv_ref):
    dp = dp_ref[0] + dp_ref[1]
    dinv = lax.rsqrt(1.0 + dp[:, 0:1])
    xw = jnp.dot(x_ref[...], w_ref[...], preferred_element_type=jnp.float32)
    g1_ref[...] = dinv * xw
    dv_ref[...] = jnp.broadcast_to(dinv, (B, 8))

  return pl.pallas_call(
      body,
      grid=GRID,
      in_specs=[_row_spec(IN_DIM), _full_spec(IN_DIM, HID), _part_spec(16)],
      out_specs=[_row_spec(HID), _row_spec(8)],
      out_shape=[jax.ShapeDtypeStruct((NP, HID), jnp.float32),
                 jax.ShapeDtypeStruct((NP, 8), jnp.float32)],
  )(x, w1, degp)


def _stage2(p1, g1, dv, b1, w2):
  def body(p_ref, g_ref, dv_ref, b_ref, w_ref, g2_ref):
    dinv = dv_ref[:, 0:1]
    h1 = jnp.maximum(dinv * (p_ref[0] + p_ref[1] + g_ref[...]) + b_ref[...],
                     0.0)
    g2_ref[...] = dinv * jnp.dot(h1, w_ref[...],
                                 preferred_element_type=jnp.float32)

  return pl.pallas_call(
      body,
      grid=GRID,
      in_specs=[_part_spec(HID), _row_spec(HID), _row_spec(8),
                _full_spec(1, HID), _full_spec(HID, HID)],
      out_specs=_row_spec(HID),
      out_shape=jax.ShapeDtypeStruct((NP, HID), jnp.float32),
  )(p1, g1, dv, b1, w2)


def _stage3(p2, g2, dv, b2, aw1):
  def body(p_ref, g_ref, dv_ref, b_ref, w_ref, g3a_ref, g3b_ref):
    dinv = dv_ref[:, 0:1]
    h = dinv * (p_ref[0] + p_ref[1] + g_ref[...]) + b_ref[...]
    hw = jnp.dot(h, w_ref[...], preferred_element_type=jnp.float32)
    g3a_ref[...] = dinv * hw
    g3b_ref[...] = dinv * h

  return pl.pallas_call(
      body,
      grid=GRID,
      in_specs=[_part_spec(HID), _row_spec(HID), _row_spec(8),
                _full_spec(1, HID), _full_spec(HID, HID)],
      out_specs=[_row_spec(HID), _row_spec(HID)],
      out_shape=[jax.ShapeDtypeStruct((NP, HID), jnp.float32),
                 jax.ShapeDtypeStruct((NP, HID), jnp.float32)],
  )(p2, g2, dv, b2, aw1)


def _stage3b(p3b, g3b, dv, sw1, sb1):
  def body(pb_ref, gb_ref, dv_ref, sw_ref, sb_ref, h_ref):
    dinv = dv_ref[:, 0:1]
    pb = pb_ref[0] + pb_ref[1]
    hpre = dinv * (pb + gb_ref[...])
    h_ref[...] = jnp.dot(hpre, sw_ref[...],
                         preferred_element_type=jnp.float32) + sb_ref[...]

  return pl.pallas_call(
      body,
      grid=GRID,
      in_specs=[_part_spec(HID), _row_spec(HID), _row_spec(8),
                _full_spec(HID, IN_DIM), _full_spec(1, IN_DIM)],
      out_specs=_row_spec(IN_DIM),
      out_shape=jax.ShapeDtypeStruct((NP, IN_DIM), jnp.float32),
  )(p3b, g3b, dv, sw1, sb1)


def _stage4(p3a, g3a, dv, ab1):
  def body(pa_ref, ga_ref, dv_ref, ab_ref, g4_ref):
    dinv = dv_ref[:, 0:1]
    pa = pa_ref[0] + pa_ref[1]
    x1 = jnp.maximum(dinv * (pa + ga_ref[...]) + ab_ref[...], 0.0)
    g4_ref[...] = dinv * x1

  return pl.pallas_call(
      body,
      grid=GRID,
      in_specs=[_part_spec(HID), _row_spec(HID), _row_spec(8),
                _full_spec(1, HID)],
      out_specs=_row_spec(HID),
      out_shape=jax.ShapeDtypeStruct((NP, HID), jnp.float32),
  )(p3a, g3a, dv, ab1)


def _stage5(p4, g4, dv, aw2, ab2):
  def body(p_ref, g_ref, dv_ref, w_ref, b_ref, x_ref):
    dinv = dv_ref[:, 0:1]
    xpre = dinv * (p_ref[0] + p_ref[1] + g_ref[...])
    x_ref[...] = jnp.dot(xpre, w_ref[...],
                         preferred_element_type=jnp.float32) + b_ref[...]

  return pl.pallas_call(
      body,
      grid=GRID,
      in_specs=[_part_spec(HID), _row_spec(HID), _row_spec(8),
                _full_spec(HID, IN_DIM), _full_spec(1, IN_DIM)],
      out_specs=_row_spec(IN_DIM),
      out_shape=jax.ShapeDtypeStruct((NP, IN_DIM), jnp.float32),
  )(p4, g4, dv, aw2, ab2)


def _adj(h):
  BM = 512

  def body(a_ref, b_ref, o_ref):
    a = a_ref[...].astype(jnp.bfloat16)
    b = b_ref[:N, :].astype(jnp.bfloat16)
    o_ref[...] = lax.dot_general(
        a, b, (((1,), (1,)), ((), ())),
        preferred_element_type=jnp.float32)

  return pl.pallas_call(
      body,
      grid=(-(-N // BM),),
      in_specs=[pl.BlockSpec((BM, IN_DIM), lambda i: (i, 0)),
                pl.BlockSpec((NP, IN_DIM), lambda i: (0, 0))],
      out_specs=pl.BlockSpec((BM, N), lambda i: (i, 0)),
      out_shape=jax.ShapeDtypeStruct((N, N), jnp.float32),
  )(h, h)


def kernel(x, edge_index, enc_W1, enc_b1, enc_W2, enc_b2,
           attr_W1, attr_b1, attr_W2, attr_b2, struct_W1, struct_b1):
  src = edge_index[0]
  dst = edge_index[1]
  pad = EP - E
  spread = jnp.arange(pad, dtype=jnp.int32)
  src_p = jnp.concatenate([src, spread % N])
  src_p = src_p.reshape(NW * CPW, CH)
  dst_p = jnp.concatenate([dst, _DUMMY_DST + spread % (NP - N)])
  dst_p = dst_p.reshape(NW * CPW, CH)

  x_p = jnp.pad(x, ((0, NP - N), (0, 0)))
  ones16 = jnp.ones((CH, DEGD), jnp.float32)
  z16 = jnp.zeros((RPS, DEGD), jnp.float32)
  z64 = jnp.zeros((RPS, HID), jnp.float32)
  b1 = enc_b1.reshape(1, HID)
  b2 = enc_b2.reshape(1, HID)
  ab1 = attr_b1.reshape(1, HID)
  ab2 = attr_b2.reshape(1, IN_DIM)
  sb1 = struct_b1.reshape(1, IN_DIM)

  degp = _make_degree()(dst_p, ones16, z16)

  g1, dv = _stage1(x_p, enc_W1, degp)
  p1 = _make_propagate(HID)(src_p, dst_p, g1, z64)
  g2 = _stage2(p1, g1, dv, b1, enc_W2)
  p2 = _make_propagate(HID)(src_p, dst_p, g2, z64)
  g3a, g3b = _stage3(p2, g2, dv, b2, attr_W1)
  p3b = _make_propagate(HID)(src_p, dst_p, g3b, z64)
  h_ = _stage3b(p3b, g3b, dv, struct_W1, sb1)
  adj = _adj(h_)
  p3a = _make_propagate(HID)(src_p, dst_p, g3a, z64)
  g4 = _stage4(p3a, g3a, dv, ab1)
  p4 = _make_propagate(HID)(src_p, dst_p, g4, z64)
  x_ = _stage5(p4, g4, dv, attr_W2, ab2)

  return (x_[:N], adj)
